# Initial kernel scaffold; baseline (speedup 1.0000x reference)
import jax
import jax.numpy as jnp
from jax.experimental import pallas as pl


def kernel(x, edge_index, Wl1, bl1, Wr1, Wl2, bl2, Wr2, Wfc, bfc):
    raise NotImplementedError("write your pallas kernel here")


# speedup vs baseline: 2.3933x; 2.3933x over previous
import functools

import jax
import jax.numpy as jnp
from jax import lax
from jax.experimental import pallas as pl
from jax.experimental.pallas import tpu as pltpu
from jax.experimental.pallas import tpu_sc as plsc

N = 10000
N_PAD = 10240
E = 160000
E_PAD = 163840
D = 256
QW = 64
NQ = 4
G = 128
NT = 16
NGROUPS = E_PAD // G
GPT = NGROUPS // NT
GPT_DEG = NGROUPS // 32
RPT = N_PAD // NT
ZR = 128


def _sc_mesh():
    return plsc.VectorSubcoreMesh(core_axis_name="c", subcore_axis_name="s")


def _agg_body(xs, src8, dst2, agg_out, src_v, dst_v, rows_v, zbuf, agg_sh,
              sem):
    c = lax.axis_index("c")
    s = lax.axis_index("s")
    zv = jnp.zeros((16,), jnp.float32)

    def zrow(i, _):
        for k in range(QW // 16):
            zbuf[i, pl.ds(16 * k, 16)] = zv
        return 0

    lax.fori_loop(0, ZR, zrow, 0)

    pltpu.sync_copy(dst2.at[pl.ds(s * GPT, GPT)], dst_v)

    for p in range(2):
        q = 2 * p + c
        row0 = s * RPT
        for j in range(RPT // ZR):
            pltpu.sync_copy(zbuf, agg_sh.at[pl.ds(row0 + j * ZR, ZR)])
        plsc.subcore_barrier()

        pltpu.sync_copy(src8.at[pl.ds(q * NGROUPS + s * GPT, GPT)], src_v)

        def step(j, _):
            pltpu.async_copy(xs.at[src_v.at[j]], rows_v, sem).wait()
            pltpu.sync_copy(rows_v, agg_sh.at[dst_v.at[j]], add=True)
            return 0

        lax.fori_loop(0, GPT, step, 0)
        plsc.subcore_barrier()

        out_row = q * N_PAD + s * RPT
        pltpu.sync_copy(agg_sh.at[pl.ds(s * RPT, RPT)],
                        agg_out.at[pl.ds(out_row, RPT)])
        plsc.subcore_barrier()


_agg = pl.kernel(
    _agg_body,
    out_type=jax.ShapeDtypeStruct((NQ * N_PAD, QW), jnp.float32),
    mesh=_sc_mesh(),
    compiler_params=pltpu.CompilerParams(use_tc_tiling_on_sc=False),
    scratch_types=[
        pltpu.VMEM((GPT, G), jnp.int32),
        pltpu.VMEM((GPT, G), jnp.int32),
        pltpu.VMEM((G, QW), jnp.float32),
        pltpu.VMEM((ZR, QW), jnp.float32),
        pltpu.VMEM_SHARED((N_PAD, QW), jnp.float32),
        pltpu.SemaphoreType.DMA,
    ],
)


def _deg_body(dst2, deg_out, dst_v, ones16, z16, deg_sh):
    c = lax.axis_index("c")
    s = lax.axis_index("s")
    zv = jnp.zeros((16,), jnp.float32)
    ov = jnp.ones((16,), jnp.float32)

    def zrow(i, _):
        z16[i, pl.ds(0, 16)] = zv
        ones16[i, pl.ds(0, 16)] = ov
        return 0

    lax.fori_loop(0, ZR, zrow, 0)

    row0 = s * RPT
    for j in range(RPT // ZR):
        pltpu.sync_copy(z16, deg_sh.at[pl.ds(row0 + j * ZR, ZR)])
    plsc.subcore_barrier()

    pltpu.sync_copy(dst2.at[pl.ds(c * (NGROUPS // 2) + s * GPT_DEG, GPT_DEG)],
                    dst_v)

    def step(j, _):
        pltpu.sync_copy(ones16, deg_sh.at[dst_v.at[j]], add=True)
        return 0

    lax.fori_loop(0, GPT_DEG, step, 0)
    plsc.subcore_barrier()

    out_row = c * N_PAD + s * RPT
    pltpu.sync_copy(deg_sh.at[pl.ds(s * RPT, RPT)],
                    deg_out.at[pl.ds(out_row, RPT)])


_deg = pl.kernel(
    _deg_body,
    out_type=jax.ShapeDtypeStruct((2 * N_PAD, 16), jnp.float32),
    mesh=_sc_mesh(),
    compiler_params=pltpu.CompilerParams(use_tc_tiling_on_sc=False),
    scratch_types=[
        pltpu.VMEM((GPT_DEG, G), jnp.int32),
        pltpu.VMEM((G, 16), jnp.float32),
        pltpu.VMEM((ZR, 16), jnp.float32),
        pltpu.VMEM_SHARED((N_PAD, 16), jnp.float32),
    ],
)


R1 = 640
R2 = 1000


def _hidden(a_ref, d_ref, x_ref, wl_ref, bl_ref, wr_ref):
    deg = d_ref[0, :, 0:1] + d_ref[1, :, 0:1]
    scale = 1.0 / jnp.maximum(deg, 1.0)
    mean = jnp.concatenate([a_ref[i] for i in range(NQ)], axis=1) * scale
    xcat = jnp.concatenate([x_ref[i] for i in range(NQ)], axis=1)
    h = (jnp.dot(mean, wl_ref[...], preferred_element_type=jnp.float32)
         + jnp.dot(xcat, wr_ref[...], preferred_element_type=jnp.float32)
         + bl_ref[...])
    return jnp.maximum(h, 0.0)


def _tc_layer_body(a_ref, d_ref, x_ref, wl_ref, bl_ref, wr_ref, o_ref):
    h = _hidden(a_ref, d_ref, x_ref, wl_ref, bl_ref, wr_ref)
    for i in range(NQ):
        o_ref[i] = h[:, i * QW:(i + 1) * QW]


def _tc_layer2_body(a_ref, d_ref, x_ref, wl_ref, bl_ref, wr_ref,
                    wfc_ref, bfc_ref, o_ref):
    h = _hidden(a_ref, d_ref, x_ref, wl_ref, bl_ref, wr_ref)
    o_ref[...] = jnp.sum(h * wfc_ref[...], axis=1, keepdims=True) + bfc_ref[...]


def _tc_layer(agg4, deg3, xs4, wlT, bl, wrT):
    grid = (N_PAD // R1,)
    specs = [
        pl.BlockSpec((NQ, R1, QW), lambda i: (0, i, 0)),
        pl.BlockSpec((2, R1, 16), lambda i: (0, i, 0)),
        pl.BlockSpec((NQ, R1, QW), lambda i: (0, i, 0)),
        pl.BlockSpec((D, D), lambda i: (0, 0)),
        pl.BlockSpec((1, D), lambda i: (0, 0)),
        pl.BlockSpec((D, D), lambda i: (0, 0)),
    ]
    return pl.pallas_call(
        _tc_layer_body,
        grid=grid,
        in_specs=specs,
        out_specs=pl.BlockSpec((NQ, R1, QW), lambda i: (0, i, 0)),
        out_shape=jax.ShapeDtypeStruct((NQ, N_PAD, QW), jnp.float32),
    )(agg4, deg3, xs4, wlT, bl, wrT)


def _tc_layer2(agg4, deg3, xs4, wlT, bl, wrT, wfc, bfc):
    grid = (N // R2,)
    specs = [
        pl.BlockSpec((NQ, R2, QW), lambda i: (0, i, 0)),
        pl.BlockSpec((2, R2, 16), lambda i: (0, i, 0)),
        pl.BlockSpec((NQ, R2, QW), lambda i: (0, i, 0)),
        pl.BlockSpec((D, D), lambda i: (0, 0)),
        pl.BlockSpec((1, D), lambda i: (0, 0)),
        pl.BlockSpec((D, D), lambda i: (0, 0)),
        pl.BlockSpec((1, D), lambda i: (0, 0)),
        pl.BlockSpec((1, 1), lambda i: (0, 0)),
    ]
    return pl.pallas_call(
        _tc_layer2_body,
        grid=grid,
        in_specs=specs,
        out_specs=pl.BlockSpec((R2, 1), lambda i: (i, 0)),
        out_shape=jax.ShapeDtypeStruct((N, 1), jnp.float32),
    )(agg4, deg3, xs4, wlT, bl, wrT, wfc, bfc)


def kernel(x, edge_index, Wl1, bl1, Wr1, Wl2, bl2, Wr2, Wfc, bfc):
    src = edge_index[0]
    dst = edge_index[1]
    pad_n = jnp.zeros((N_PAD - N, QW), jnp.float32)
    xs = jnp.concatenate(
        [t for i in range(NQ) for t in (x[:, i * QW:(i + 1) * QW], pad_n)],
        axis=0)
    src_p = jnp.concatenate([src, jnp.zeros((E_PAD - E,), jnp.int32)])
    dst_p = jnp.concatenate([dst, jnp.full((E_PAD - E,), N, jnp.int32)])
    src2 = src_p.reshape(NGROUPS, G)
    src8 = jnp.concatenate([src2 + q * N_PAD for q in range(NQ)], axis=0)
    dst2 = dst_p.reshape(NGROUPS, G)

    degm = _deg(dst2)
    deg3 = degm.reshape(2, N_PAD, 16)
    agg1 = _agg(xs, src8, dst2).reshape(NQ, N_PAD, QW)
    h1 = _tc_layer(agg1, deg3, xs.reshape(NQ, N_PAD, QW),
                   Wl1.T, bl1.reshape(1, D), Wr1.T)

    agg2 = _agg(h1.reshape(NQ * N_PAD, QW), src8, dst2).reshape(
        NQ, N_PAD, QW)
    out = _tc_layer2(agg2, deg3, h1, Wl2.T, bl2.reshape(1, D), Wr2.T,
                     Wfc, bfc.reshape(1, 1))
    return out.reshape(N)


# speedup vs baseline: 3.0865x; 1.2897x over previous
import functools

import jax
import jax.numpy as jnp
from jax import lax
from jax.experimental import pallas as pl
from jax.experimental.pallas import tpu as pltpu
from jax.experimental.pallas import tpu_sc as plsc

N = 10000
N_PAD = 10240
E = 160000
E_PAD = 163840
D = 256
QW = 64
NQ = 4
G = 128
NT = 16
NGROUPS = E_PAD // G
GPT = NGROUPS // NT
GPT_DEG = NGROUPS // 32
RPT = N_PAD // NT
ZR = 128


def _sc_mesh():
    return plsc.VectorSubcoreMesh(core_axis_name="c", subcore_axis_name="s")


NB = 4
LEAD = 2
CH = 8
NCHUNK = GPT // CH


def _agg_body(xs, src8, dst2, agg_out, *rest):
    src_c = rest[0:2]
    dst_c = rest[2:4]
    agg_sh = rest[4]
    rows = rest[5:5 + NB]
    gsem = rest[5 + NB:5 + 2 * NB]
    ssem = rest[5 + 2 * NB:5 + 3 * NB]
    isem_s, isem_d = rest[5 + 3 * NB:]
    c = lax.axis_index("c")
    s = lax.axis_index("s")
    zv = jnp.zeros((16,), jnp.float32)
    dbase = s * GPT

    def zero_rows0(i, _):
        for k in range(QW // 16):
            rows[0][i, pl.ds(16 * k, 16)] = zv
        return 0

    def fire_gather(idx_row, b):
        pltpu.async_copy(xs.at[idx_row], rows[b], gsem[b])

    def wait_gather(b):
        pltpu.make_async_copy(xs.at[src_c[0].at[0]], rows[b],
                              gsem[b]).wait()

    def fire_scatter(idx_row, b):
        pltpu.async_copy(rows[b], agg_sh.at[idx_row], ssem[b], add=True)

    def wait_scatter(b):
        pltpu.make_async_copy(rows[b], agg_sh.at[dst_c[0].at[0]],
                              ssem[b]).wait()

    for p in range(2):
        q = 2 * p + c
        qbase = q * NGROUPS + s * GPT

        def load_idx_sync(kc, pp):
            pltpu.sync_copy(src8.at[pl.ds(qbase + kc * CH, CH)], src_c[pp])
            pltpu.sync_copy(dst2.at[pl.ds(dbase + kc * CH, CH)], dst_c[pp])

        def fire_prefetch(kc, pp):
            pltpu.async_copy(src8.at[pl.ds(qbase + kc * CH, CH)],
                             src_c[pp], isem_s)
            pltpu.async_copy(dst2.at[pl.ds(dbase + kc * CH, CH)],
                             dst_c[pp], isem_d)

        def wait_idx(pp):
            pltpu.make_async_copy(src8.at[pl.ds(qbase, CH)], src_c[pp],
                                  isem_s).wait()
            pltpu.make_async_copy(dst2.at[pl.ds(dbase, CH)], dst_c[pp],
                                  isem_d).wait()

        def chunk_slots(k, pk, first_chunk, last_chunk, prefetch):
            if prefetch:
                fire_prefetch(k + 1, 1 - pk)
            for r in range(CH):
                b = r % NB
                bg = (r + LEAD) % NB
                if not (first_chunk and r < LEAD):
                    wait_scatter(bg)
                if r == CH - LEAD and not last_chunk:
                    wait_idx(1 - pk)
                if not (last_chunk and r >= CH - LEAD):
                    p2 = (pk + (1 if r >= CH - LEAD else 0)) % 2
                    fire_gather(src_c[p2].at[(r + LEAD) % CH], bg)
                wait_gather(b)
                fire_scatter(dst_c[pk].at[r], b)

        lax.fori_loop(0, G, zero_rows0, 0)
        row0 = s * RPT
        for j in range(RPT // G):
            pltpu.sync_copy(rows[0], agg_sh.at[pl.ds(row0 + j * G, G)])
        plsc.subcore_barrier()

        load_idx_sync(0, 0)
        fire_prefetch(1, 1)
        for g in range(LEAD):
            fire_gather(src_c[0].at[g], g % NB)

        chunk_slots(0, 0, True, False, False)

        def pair_body(kk, _):
            chunk_slots(1 + 2 * kk, 1, False, False, True)
            chunk_slots(2 + 2 * kk, 0, False, False, True)
            return 0

        lax.fori_loop(0, (NCHUNK - 2) // 2, pair_body, 0)
        chunk_slots(NCHUNK - 1, (NCHUNK - 1) % 2, False, True, False)

        for g in range(GPT - LEAD, GPT):
            wait_scatter(g % NB)
        plsc.subcore_barrier()

        out_row = q * N_PAD + s * RPT
        pltpu.sync_copy(agg_sh.at[pl.ds(s * RPT, RPT)],
                        agg_out.at[pl.ds(out_row, RPT)])
        plsc.subcore_barrier()


_agg = pl.kernel(
    _agg_body,
    out_type=jax.ShapeDtypeStruct((NQ * N_PAD, QW), jnp.float32),
    mesh=_sc_mesh(),
    compiler_params=pltpu.CompilerParams(use_tc_tiling_on_sc=False),
    scratch_types=(
        [pltpu.VMEM((CH, G), jnp.int32) for _ in range(2)]
        + [pltpu.VMEM((CH, G), jnp.int32) for _ in range(2)]
        + [pltpu.VMEM_SHARED((N_PAD, QW), jnp.float32)]
        + [pltpu.VMEM((G, QW), jnp.float32) for _ in range(NB)]
        + [pltpu.SemaphoreType.DMA for _ in range(2 * NB + 2)]
    ),
)


def _deg_body(dst2, deg_out, dst_v, ones16, z16, deg_sh):
    c = lax.axis_index("c")
    s = lax.axis_index("s")
    zv = jnp.zeros((16,), jnp.float32)
    ov = jnp.ones((16,), jnp.float32)

    def zrow(i, _):
        z16[i, pl.ds(0, 16)] = zv
        ones16[i, pl.ds(0, 16)] = ov
        return 0

    lax.fori_loop(0, ZR, zrow, 0)

    row0 = s * RPT
    for j in range(RPT // ZR):
        pltpu.sync_copy(z16, deg_sh.at[pl.ds(row0 + j * ZR, ZR)])
    plsc.subcore_barrier()

    pltpu.sync_copy(dst2.at[pl.ds(c * (NGROUPS // 2) + s * GPT_DEG, GPT_DEG)],
                    dst_v)

    def step(j, _):
        pltpu.sync_copy(ones16, deg_sh.at[dst_v.at[j]], add=True)
        return 0

    lax.fori_loop(0, GPT_DEG, step, 0)
    plsc.subcore_barrier()

    out_row = c * N_PAD + s * RPT
    pltpu.sync_copy(deg_sh.at[pl.ds(s * RPT, RPT)],
                    deg_out.at[pl.ds(out_row, RPT)])


_deg = pl.kernel(
    _deg_body,
    out_type=jax.ShapeDtypeStruct((2 * N_PAD, 16), jnp.float32),
    mesh=_sc_mesh(),
    compiler_params=pltpu.CompilerParams(use_tc_tiling_on_sc=False),
    scratch_types=[
        pltpu.VMEM((GPT_DEG, G), jnp.int32),
        pltpu.VMEM((G, 16), jnp.float32),
        pltpu.VMEM((ZR, 16), jnp.float32),
        pltpu.VMEM_SHARED((N_PAD, 16), jnp.float32),
    ],
)


R1 = 640
R2 = 1000


def _hidden(a_ref, d_ref, x_ref, wl_ref, bl_ref, wr_ref):
    deg = d_ref[0, :, 0:1] + d_ref[1, :, 0:1]
    scale = 1.0 / jnp.maximum(deg, 1.0)
    mean = jnp.concatenate([a_ref[i] for i in range(NQ)], axis=1) * scale
    xcat = jnp.concatenate([x_ref[i] for i in range(NQ)], axis=1)
    h = (jnp.dot(mean, wl_ref[...], preferred_element_type=jnp.float32)
         + jnp.dot(xcat, wr_ref[...], preferred_element_type=jnp.float32)
         + bl_ref[...])
    return jnp.maximum(h, 0.0)


def _tc_layer_body(a_ref, d_ref, x_ref, wl_ref, bl_ref, wr_ref, o_ref):
    h = _hidden(a_ref, d_ref, x_ref, wl_ref, bl_ref, wr_ref)
    for i in range(NQ):
        o_ref[i] = h[:, i * QW:(i + 1) * QW]


def _tc_layer2_body(a_ref, d_ref, x_ref, wl_ref, bl_ref, wr_ref,
                    wfc_ref, bfc_ref, o_ref):
    h = _hidden(a_ref, d_ref, x_ref, wl_ref, bl_ref, wr_ref)
    o_ref[...] = jnp.sum(h * wfc_ref[...], axis=1, keepdims=True) + bfc_ref[...]


def _tc_layer(agg4, deg3, xs4, wlT, bl, wrT):
    grid = (N_PAD // R1,)
    specs = [
        pl.BlockSpec((NQ, R1, QW), lambda i: (0, i, 0)),
        pl.BlockSpec((2, R1, 16), lambda i: (0, i, 0)),
        pl.BlockSpec((NQ, R1, QW), lambda i: (0, i, 0)),
        pl.BlockSpec((D, D), lambda i: (0, 0)),
        pl.BlockSpec((1, D), lambda i: (0, 0)),
        pl.BlockSpec((D, D), lambda i: (0, 0)),
    ]
    return pl.pallas_call(
        _tc_layer_body,
        grid=grid,
        in_specs=specs,
        out_specs=pl.BlockSpec((NQ, R1, QW), lambda i: (0, i, 0)),
        out_shape=jax.ShapeDtypeStruct((NQ, N_PAD, QW), jnp.float32),
    )(agg4, deg3, xs4, wlT, bl, wrT)


def _tc_layer2(agg4, deg3, xs4, wlT, bl, wrT, wfc, bfc):
    grid = (N // R2,)
    specs = [
        pl.BlockSpec((NQ, R2, QW), lambda i: (0, i, 0)),
        pl.BlockSpec((2, R2, 16), lambda i: (0, i, 0)),
        pl.BlockSpec((NQ, R2, QW), lambda i: (0, i, 0)),
        pl.BlockSpec((D, D), lambda i: (0, 0)),
        pl.BlockSpec((1, D), lambda i: (0, 0)),
        pl.BlockSpec((D, D), lambda i: (0, 0)),
        pl.BlockSpec((1, D), lambda i: (0, 0)),
        pl.BlockSpec((1, 1), lambda i: (0, 0)),
    ]
    return pl.pallas_call(
        _tc_layer2_body,
        grid=grid,
        in_specs=specs,
        out_specs=pl.BlockSpec((R2, 1), lambda i: (i, 0)),
        out_shape=jax.ShapeDtypeStruct((N, 1), jnp.float32),
    )(agg4, deg3, xs4, wlT, bl, wrT, wfc, bfc)


def kernel(x, edge_index, Wl1, bl1, Wr1, Wl2, bl2, Wr2, Wfc, bfc):
    src = edge_index[0]
    dst = edge_index[1]
    pad_n = jnp.zeros((N_PAD - N, QW), jnp.float32)
    xs = jnp.concatenate(
        [t for i in range(NQ) for t in (x[:, i * QW:(i + 1) * QW], pad_n)],
        axis=0)
    src_p = jnp.concatenate([src, jnp.zeros((E_PAD - E,), jnp.int32)])
    dst_p = jnp.concatenate([dst, jnp.full((E_PAD - E,), N, jnp.int32)])
    src2 = src_p.reshape(NGROUPS, G)
    src8 = jnp.concatenate([src2 + q * N_PAD for q in range(NQ)], axis=0)
    dst2 = dst_p.reshape(NGROUPS, G)

    degm = _deg(dst2)
    deg3 = degm.reshape(2, N_PAD, 16)
    agg1 = _agg(xs, src8, dst2).reshape(NQ, N_PAD, QW)
    h1 = _tc_layer(agg1, deg3, xs.reshape(NQ, N_PAD, QW),
                   Wl1.T, bl1.reshape(1, D), Wr1.T)

    agg2 = _agg(h1.reshape(NQ * N_PAD, QW), src8, dst2).reshape(
        NQ, N_PAD, QW)
    out = _tc_layer2(agg2, deg3, h1, Wl2.T, bl2.reshape(1, D), Wr2.T,
                     Wfc, bfc.reshape(1, 1))
    return out.reshape(N)


# speedup vs baseline: 3.3409x; 1.0824x over previous
import functools

import jax
import jax.numpy as jnp
from jax import lax
from jax.experimental import pallas as pl
from jax.experimental.pallas import tpu as pltpu
from jax.experimental.pallas import tpu_sc as plsc

N = 10000
N_PAD = 10240
E = 160000
E_PAD = 163840
D = 256
QW = 64
NQ = 4
G = 64
NT = 16
NGROUPS = E_PAD // G
GPT = NGROUPS // NT
GPT_DEG = NGROUPS // 32
RPT = N_PAD // NT
ZR = 128


def _sc_mesh():
    return plsc.VectorSubcoreMesh(core_axis_name="c", subcore_axis_name="s")


NB = 8
LEAD = 5
CH = 8
NCHUNK = GPT // CH


def _agg_body(xs, src8, dst2, agg_out, *rest):
    src_c = rest[0:2]
    dst_c = rest[2:4]
    agg_sh = rest[4]
    rows = rest[5:5 + NB]
    gsem = rest[5 + NB:5 + 2 * NB]
    ssem = rest[5 + 2 * NB:5 + 3 * NB]
    isem_s, isem_d = rest[5 + 3 * NB:]
    c = lax.axis_index("c")
    s = lax.axis_index("s")
    zv = jnp.zeros((16,), jnp.float32)
    dbase = s * GPT

    def zero_rows0(i, _):
        for k in range(QW // 16):
            rows[0][i, pl.ds(16 * k, 16)] = zv
        return 0

    def fire_gather(idx_row, b):
        pltpu.async_copy(xs.at[idx_row], rows[b], gsem[b])

    def wait_gather(b):
        pltpu.make_async_copy(xs.at[src_c[0].at[0]], rows[b],
                              gsem[b]).wait()

    def fire_scatter(idx_row, b):
        pltpu.async_copy(rows[b], agg_sh.at[idx_row], ssem[b], add=True)

    def wait_scatter(b):
        pltpu.make_async_copy(rows[b], agg_sh.at[dst_c[0].at[0]],
                              ssem[b]).wait()

    for p in range(2):
        q = 2 * p + c
        qbase = q * NGROUPS + s * GPT

        def load_idx_sync(kc, pp):
            pltpu.sync_copy(src8.at[pl.ds(qbase + kc * CH, CH)], src_c[pp])
            pltpu.sync_copy(dst2.at[pl.ds(dbase + kc * CH, CH)], dst_c[pp])

        def fire_prefetch(kc, pp):
            pltpu.async_copy(src8.at[pl.ds(qbase + kc * CH, CH)],
                             src_c[pp], isem_s)
            pltpu.async_copy(dst2.at[pl.ds(dbase + kc * CH, CH)],
                             dst_c[pp], isem_d)

        def wait_idx(pp):
            pltpu.make_async_copy(src8.at[pl.ds(qbase, CH)], src_c[pp],
                                  isem_s).wait()
            pltpu.make_async_copy(dst2.at[pl.ds(dbase, CH)], dst_c[pp],
                                  isem_d).wait()

        def chunk_slots(k, pk, first_chunk, last_chunk, prefetch):
            if prefetch:
                fire_prefetch(k + 1, 1 - pk)
            for r in range(CH):
                b = r % NB
                bg = (r + LEAD) % NB
                if not (first_chunk and r < NB - LEAD):
                    wait_scatter(bg)
                if r == CH - LEAD and not last_chunk:
                    wait_idx(1 - pk)
                if not (last_chunk and r >= CH - LEAD):
                    p2 = (pk + (1 if r >= CH - LEAD else 0)) % 2
                    fire_gather(src_c[p2].at[(r + LEAD) % CH], bg)
                wait_gather(b)
                fire_scatter(dst_c[pk].at[r], b)

        lax.fori_loop(0, G, zero_rows0, 0)
        row0 = s * RPT
        for j in range(RPT // G):
            pltpu.sync_copy(rows[0], agg_sh.at[pl.ds(row0 + j * G, G)])
        plsc.subcore_barrier()

        load_idx_sync(0, 0)
        fire_prefetch(1, 1)
        for g in range(LEAD):
            fire_gather(src_c[0].at[g], g % NB)

        chunk_slots(0, 0, True, False, False)

        def pair_body(kk, _):
            chunk_slots(1 + 2 * kk, 1, False, False, True)
            chunk_slots(2 + 2 * kk, 0, False, False, True)
            return 0

        lax.fori_loop(0, (NCHUNK - 2) // 2, pair_body, 0)
        chunk_slots(NCHUNK - 1, (NCHUNK - 1) % 2, False, True, False)

        for g in range(GPT - (NB - LEAD), GPT):
            wait_scatter(g % NB)
        plsc.subcore_barrier()

        out_row = q * N_PAD + s * RPT
        pltpu.sync_copy(agg_sh.at[pl.ds(s * RPT, RPT)],
                        agg_out.at[pl.ds(out_row, RPT)])
        plsc.subcore_barrier()


_agg = pl.kernel(
    _agg_body,
    out_type=jax.ShapeDtypeStruct((NQ * N_PAD, QW), jnp.float32),
    mesh=_sc_mesh(),
    compiler_params=pltpu.CompilerParams(use_tc_tiling_on_sc=False),
    scratch_types=(
        [pltpu.VMEM((CH, G), jnp.int32) for _ in range(2)]
        + [pltpu.VMEM((CH, G), jnp.int32) for _ in range(2)]
        + [pltpu.VMEM_SHARED((N_PAD, QW), jnp.float32)]
        + [pltpu.VMEM((G, QW), jnp.float32) for _ in range(NB)]
        + [pltpu.SemaphoreType.DMA for _ in range(2 * NB + 2)]
    ),
)


def _deg_body(dst2, deg_out, dst_v, ones16, z16, deg_sh):
    c = lax.axis_index("c")
    s = lax.axis_index("s")
    zv = jnp.zeros((16,), jnp.float32)
    ov = jnp.ones((16,), jnp.float32)

    def zrow(i, _):
        z16[i, pl.ds(0, 16)] = zv
        ones16[jnp.minimum(i, G - 1), pl.ds(0, 16)] = ov
        return 0

    lax.fori_loop(0, ZR, zrow, 0)

    row0 = s * RPT
    for j in range(RPT // ZR):
        pltpu.sync_copy(z16, deg_sh.at[pl.ds(row0 + j * ZR, ZR)])
    plsc.subcore_barrier()

    pltpu.sync_copy(dst2.at[pl.ds(c * (NGROUPS // 2) + s * GPT_DEG, GPT_DEG)],
                    dst_v)

    def step(j, _):
        pltpu.sync_copy(ones16, deg_sh.at[dst_v.at[j]], add=True)
        return 0

    lax.fori_loop(0, GPT_DEG, step, 0)
    plsc.subcore_barrier()

    out_row = c * N_PAD + s * RPT
    pltpu.sync_copy(deg_sh.at[pl.ds(s * RPT, RPT)],
                    deg_out.at[pl.ds(out_row, RPT)])


_deg = pl.kernel(
    _deg_body,
    out_type=jax.ShapeDtypeStruct((2 * N_PAD, 16), jnp.float32),
    mesh=_sc_mesh(),
    compiler_params=pltpu.CompilerParams(use_tc_tiling_on_sc=False),
    scratch_types=[
        pltpu.VMEM((GPT_DEG, G), jnp.int32),
        pltpu.VMEM((G, 16), jnp.float32),
        pltpu.VMEM((ZR, 16), jnp.float32),
        pltpu.VMEM_SHARED((N_PAD, 16), jnp.float32),
    ],
)


R1 = 640
R2 = 1000


def _hidden(a_ref, d_ref, x_ref, wl_ref, bl_ref, wr_ref):
    deg = d_ref[0, :, 0:1] + d_ref[1, :, 0:1]
    scale = 1.0 / jnp.maximum(deg, 1.0)
    mean = jnp.concatenate([a_ref[i] for i in range(NQ)], axis=1) * scale
    xcat = jnp.concatenate([x_ref[i] for i in range(NQ)], axis=1)
    h = (jnp.dot(mean, wl_ref[...], preferred_element_type=jnp.float32)
         + jnp.dot(xcat, wr_ref[...], preferred_element_type=jnp.float32)
         + bl_ref[...])
    return jnp.maximum(h, 0.0)


def _tc_layer_body(a_ref, d_ref, x_ref, wl_ref, bl_ref, wr_ref, o_ref):
    h = _hidden(a_ref, d_ref, x_ref, wl_ref, bl_ref, wr_ref)
    for i in range(NQ):
        o_ref[i] = h[:, i * QW:(i + 1) * QW]


def _tc_layer2_body(a_ref, d_ref, x_ref, wl_ref, bl_ref, wr_ref,
                    wfc_ref, bfc_ref, o_ref):
    h = _hidden(a_ref, d_ref, x_ref, wl_ref, bl_ref, wr_ref)
    o_ref[...] = jnp.sum(h * wfc_ref[...], axis=1, keepdims=True) + bfc_ref[...]


def _tc_layer(agg4, deg3, xs4, wlT, bl, wrT):
    grid = (N_PAD // R1,)
    specs = [
        pl.BlockSpec((NQ, R1, QW), lambda i: (0, i, 0)),
        pl.BlockSpec((2, R1, 16), lambda i: (0, i, 0)),
        pl.BlockSpec((NQ, R1, QW), lambda i: (0, i, 0)),
        pl.BlockSpec((D, D), lambda i: (0, 0)),
        pl.BlockSpec((1, D), lambda i: (0, 0)),
        pl.BlockSpec((D, D), lambda i: (0, 0)),
    ]
    return pl.pallas_call(
        _tc_layer_body,
        grid=grid,
        in_specs=specs,
        out_specs=pl.BlockSpec((NQ, R1, QW), lambda i: (0, i, 0)),
        out_shape=jax.ShapeDtypeStruct((NQ, N_PAD, QW), jnp.float32),
    )(agg4, deg3, xs4, wlT, bl, wrT)


def _tc_layer2(agg4, deg3, xs4, wlT, bl, wrT, wfc, bfc):
    grid = (N // R2,)
    specs = [
        pl.BlockSpec((NQ, R2, QW), lambda i: (0, i, 0)),
        pl.BlockSpec((2, R2, 16), lambda i: (0, i, 0)),
        pl.BlockSpec((NQ, R2, QW), lambda i: (0, i, 0)),
        pl.BlockSpec((D, D), lambda i: (0, 0)),
        pl.BlockSpec((1, D), lambda i: (0, 0)),
        pl.BlockSpec((D, D), lambda i: (0, 0)),
        pl.BlockSpec((1, D), lambda i: (0, 0)),
        pl.BlockSpec((1, 1), lambda i: (0, 0)),
    ]
    return pl.pallas_call(
        _tc_layer2_body,
        grid=grid,
        in_specs=specs,
        out_specs=pl.BlockSpec((R2, 1), lambda i: (i, 0)),
        out_shape=jax.ShapeDtypeStruct((N, 1), jnp.float32),
    )(agg4, deg3, xs4, wlT, bl, wrT, wfc, bfc)


def kernel(x, edge_index, Wl1, bl1, Wr1, Wl2, bl2, Wr2, Wfc, bfc):
    src = edge_index[0]
    dst = edge_index[1]
    pad_n = jnp.zeros((N_PAD - N, QW), jnp.float32)
    xs = jnp.concatenate(
        [t for i in range(NQ) for t in (x[:, i * QW:(i + 1) * QW], pad_n)],
        axis=0)
    src_p = jnp.concatenate([src, jnp.zeros((E_PAD - E,), jnp.int32)])
    dst_p = jnp.concatenate([dst, jnp.full((E_PAD - E,), N, jnp.int32)])
    src2 = src_p.reshape(NGROUPS, G)
    src8 = jnp.concatenate([src2 + q * N_PAD for q in range(NQ)], axis=0)
    dst2 = dst_p.reshape(NGROUPS, G)
    xs4 = xs.reshape(NQ, N_PAD, QW)

    degm = _deg(dst2)
    deg3 = degm.reshape(2, N_PAD, 16)
    agg1 = _agg(xs, src8, dst2).reshape(NQ, N_PAD, QW)
    h1 = _tc_layer(agg1, deg3, xs4, Wl1.T, bl1.reshape(1, D), Wr1.T)

    agg2 = _agg(h1.reshape(NQ * N_PAD, QW), src8, dst2).reshape(
        NQ, N_PAD, QW)
    out = _tc_layer2(agg2, deg3, h1, Wl2.T, bl2.reshape(1, D), Wr2.T,
                     Wfc, bfc.reshape(1, 1))
    return out.reshape(N)
